# baseline (speedup 1.0000x reference)
import functools

import jax
import jax.numpy as jnp
from jax import lax
from jax.experimental import pallas as pl
from jax.experimental.pallas import tpu as pltpu
from jax.experimental.pallas import tpu_sc as plsc

F = 16
B = 4096
D = 64
K = 1024
BETA = 0.25
TB = 2048
DP = 128

NC = 2
NS = 16
NW = NC * NS
FB = F * B
GN = FB // 2
RPW = GN // NW
CH = 128
NCH = RPW // CH
NBUF = 2


def _score_body(x_ref, w_ref, wsq_ref, idx_ref, wt_ref, acc_ref):
    x = x_ref[0]
    w = w_ref[0]
    xsq = jnp.sum(x * x, axis=1, keepdims=True)
    wsq = wsq_ref[0]
    mm2 = jnp.dot(x + x, w, preferred_element_type=jnp.float32)
    scores = (xsq - mm2) + wsq
    minv = jnp.min(scores, axis=1, keepdims=True)
    iota = lax.broadcasted_iota(jnp.int32, (TB, K), 1).astype(jnp.float32)
    cand = jnp.where(scores == minv, iota, float(K))
    idx = jnp.min(cand, axis=1, keepdims=True).astype(jnp.int32)
    f = pl.program_id(0)
    b = pl.program_id(1)
    idx_ref[0] = idx + f * K

    @pl.when(b == 0)
    def _():
        wt_ref[0] = jnp.pad(w.T, ((0, 0), (0, DP - D)))

    tile_sum = jnp.sum(minv).reshape(1, 1)

    @pl.when((f == 0) & (b == 0))
    def _():
        acc_ref[...] = jnp.zeros((1, 1), jnp.float32)

    acc_ref[...] += tile_sum


def _gather_body(table_hbm, idxf_hbm, out_hbm, idx_v, rows_v, sem0, sem1):
    wid = lax.axis_index("s") * NC + lax.axis_index("c")
    base = wid * RPW
    pltpu.sync_copy(idxf_hbm.at[pl.ds(base, RPW)], idx_v)
    sems = (sem0, sem1)

    def fire(i):
        j = i % NBUF
        return pltpu.async_copy(
            table_hbm.at[idx_v.at[pl.ds(i * CH, CH)]], rows_v.at[j], sems[j])

    pending = [fire(0)]
    for i in range(NCH):
        if i + 1 < NCH:
            pending.append(fire(i + 1))
        pending[i].wait()
        pltpu.sync_copy(rows_v.at[i % NBUF, :, pl.ds(0, D)],
                        out_hbm.at[pl.ds(base + i * CH, CH)])


_sc_gather = functools.partial(
    pl.kernel,
    mesh=plsc.VectorSubcoreMesh(core_axis_name="c", subcore_axis_name="s"),
    compiler_params=pltpu.CompilerParams(use_tc_tiling_on_sc=False),
    out_type=jax.ShapeDtypeStruct((GN, D), jnp.float32),
    scratch_types=[
        pltpu.VMEM((RPW,), jnp.int32),
        pltpu.VMEM((NBUF, CH, DP), jnp.float32),
        pltpu.SemaphoreType.DMA,
        pltpu.SemaphoreType.DMA,
    ],
)(_gather_body)


FH = F // 2


def _score_half(inputs, wsq, w, off):
    return pl.pallas_call(
        _score_body,
        grid=(FH, B // TB),
        in_specs=[
            pl.BlockSpec((1, TB, D), lambda f, b: (off + f, b, 0)),
            pl.BlockSpec((1, D, K), lambda f, b: (off + f, 0, 0)),
            pl.BlockSpec((1, 1, K), lambda f, b: (off + f, 0, 0)),
        ],
        out_specs=[
            pl.BlockSpec((1, TB, 1), lambda f, b: (f, b, 0)),
            pl.BlockSpec((1, K, DP), lambda f, b: (f, 0, 0)),
            pl.BlockSpec((1, 1), lambda f, b: (0, 0)),
        ],
        out_shape=[
            jax.ShapeDtypeStruct((FH, B, 1), jnp.int32),
            jax.ShapeDtypeStruct((FH, K, DP), jnp.float32),
            jax.ShapeDtypeStruct((1, 1), jnp.float32),
        ],
    )(inputs, w, wsq)


def kernel(inputs, w):
    wsq = jnp.sum(w * w, axis=1, keepdims=True)
    idx_a, wt_a, acc_a = _score_half(inputs, wsq, w, 0)
    q_a = _sc_gather(wt_a.reshape(FH * K, DP), idx_a.reshape(GN))
    idx_b, wt_b, acc_b = _score_half(inputs, wsq, w, FH)
    q_b = _sc_gather(wt_b.reshape(FH * K, DP), idx_b.reshape(GN))
    out = jnp.concatenate([q_a, q_b], axis=0).reshape(F, B, D)
    m = (acc_a[0, 0] + acc_b[0, 0]) / float(F * B * D)
    loss = m + BETA * m
    return (out, loss)


# speedup vs baseline: 1.1286x; 1.1286x over previous
import functools

import jax
import jax.numpy as jnp
from jax import lax
from jax.experimental import pallas as pl
from jax.experimental.pallas import tpu as pltpu
from jax.experimental.pallas import tpu_sc as plsc

F = 16
B = 4096
D = 64
K = 1024
BETA = 0.25
TB = 4096
DP = 128

NC = 2
NS = 16
NW = NC * NS
FB = F * B
RPW = FB // NW
CH = 128
NCH = RPW // CH
NBUF = 2


def _score_body(x_ref, w_ref, wsq_ref, idx_ref, wt_ref, acc_ref):
    x = x_ref[0]
    w = w_ref[0]
    xsq = jnp.sum(x * x, axis=1, keepdims=True)
    wsq = wsq_ref[0]
    mm2 = jnp.dot(x + x, w, preferred_element_type=jnp.float32)
    scores = (xsq - mm2) + wsq
    minv = jnp.min(scores, axis=1, keepdims=True)
    iota = lax.broadcasted_iota(jnp.int32, (TB, K), 1).astype(jnp.float32)
    cand = jnp.where(scores == minv, iota, float(K))
    idx = jnp.min(cand, axis=1, keepdims=True).astype(jnp.int32)
    f = pl.program_id(0)
    b = pl.program_id(1)
    idx_ref[0] = idx + f * K

    @pl.when(b == 0)
    def _():
        wt_ref[0] = jnp.pad(w.T, ((0, 0), (0, DP - D)))

    tile_sum = jnp.sum(minv).reshape(1, 1)

    @pl.when((f == 0) & (b == 0))
    def _():
        acc_ref[...] = jnp.zeros((1, 1), jnp.float32)

    acc_ref[...] += tile_sum


def _gather_body(table_hbm, idxf_hbm, out_hbm, idx_v, rows_v, sem0, sem1):
    wid = lax.axis_index("s") * NC + lax.axis_index("c")
    base = wid * RPW
    pltpu.sync_copy(idxf_hbm.at[pl.ds(base, RPW)], idx_v)
    sems = (sem0, sem1)

    def fire(i):
        j = i % NBUF
        return pltpu.async_copy(
            table_hbm.at[idx_v.at[pl.ds(i * CH, CH)]], rows_v.at[j], sems[j])

    pending = [fire(0)]
    for i in range(NCH):
        if i + 1 < NCH:
            pending.append(fire(i + 1))
        pending[i].wait()
        pltpu.sync_copy(rows_v.at[i % NBUF, :, pl.ds(0, D)],
                        out_hbm.at[pl.ds(base + i * CH, CH)])


_sc_gather = functools.partial(
    pl.kernel,
    mesh=plsc.VectorSubcoreMesh(core_axis_name="c", subcore_axis_name="s"),
    compiler_params=pltpu.CompilerParams(use_tc_tiling_on_sc=False),
    out_type=jax.ShapeDtypeStruct((FB, D), jnp.float32),
    scratch_types=[
        pltpu.VMEM((RPW,), jnp.int32),
        pltpu.VMEM((NBUF, CH, DP), jnp.float32),
        pltpu.SemaphoreType.DMA,
        pltpu.SemaphoreType.DMA,
    ],
)(_gather_body)


def kernel(inputs, w):
    idx3, wt, acc = pl.pallas_call(
        _score_body,
        grid=(F, B // TB),
        in_specs=[
            pl.BlockSpec((1, TB, D), lambda f, b: (f, b, 0)),
            pl.BlockSpec((1, D, K), lambda f, b: (f, 0, 0)),
            pl.BlockSpec((1, 1, K), lambda f, b: (f, 0, 0)),
        ],
        out_specs=[
            pl.BlockSpec((1, TB, 1), lambda f, b: (f, b, 0)),
            pl.BlockSpec((1, K, DP), lambda f, b: (f, 0, 0)),
            pl.BlockSpec((1, 1), lambda f, b: (0, 0)),
        ],
        out_shape=[
            jax.ShapeDtypeStruct((F, B, 1), jnp.int32),
            jax.ShapeDtypeStruct((F, K, DP), jnp.float32),
            jax.ShapeDtypeStruct((1, 1), jnp.float32),
        ],
    )(inputs, w, jnp.sum(w * w, axis=1, keepdims=True))
    q = _sc_gather(wt.reshape(F * K, DP), idx3.reshape(FB))
    out = q.reshape(F, B, D)
    m = acc[0, 0] / float(F * B * D)
    loss = m + BETA * m
    return (out, loss)
